# Initial kernel scaffold; baseline (speedup 1.0000x reference)
import jax
import jax.numpy as jnp
from jax.experimental import pallas as pl


def kernel(input_ids, token_type_ids, item_position_ids, word_emb, pos_emb, type_emb, item_emb, ln_gamma, ln_beta):
    raise NotImplementedError("write your pallas kernel here")


# speedup vs baseline: 1.8324x; 1.8324x over previous
import functools

import jax
import jax.numpy as jnp
from jax import lax
from jax.experimental import pallas as pl
from jax.experimental.pallas import tpu as pltpu
from jax.experimental.pallas import tpu_sc as plsc

VOCAB = 50265
HIDDEN = 768
PAD = 1
EPS = 1e-12
B, S = 4, 2048
TOK = B * S

NUM_WORKERS = 32
PER_W = TOK // NUM_WORKERS
CHUNK = 64
NCHUNK = PER_W // CHUNK
LANES = 16
HVECS = HIDDEN // LANES


def _pos_ids_body(ids_ref, out_ref):
    mask = (ids_ref[...] != PAD).astype(jnp.int32)
    c = mask
    k = 1
    while k < S:
        shifted = jnp.concatenate(
            [jnp.zeros((B, k), jnp.int32), c[:, : S - k]], axis=1
        )
        c = c + shifted
        k *= 2
    out_ref[...] = c * mask + PAD


def _position_ids(input_ids):
    return pl.pallas_call(
        _pos_ids_body,
        out_shape=jax.ShapeDtypeStruct((B, S), jnp.int32),
    )(input_ids)


def _sc_body(wids_hbm, pids_hbm, wtab_hbm, ptab_hbm, out_hbm,
             idx_w, idx_p, buf_w, buf_p, sem_w, sem_p):
    cid = lax.axis_index("c")
    sid = lax.axis_index("s")
    wid = cid * 16 + sid
    base = wid * PER_W

    def chunk_body(j, _):
        start = base + j * CHUNK
        pltpu.sync_copy(wids_hbm.at[pl.ds(start, CHUNK)], idx_w)
        cp_w = pltpu.async_copy(wtab_hbm.at[idx_w], buf_w, sem_w)
        pltpu.sync_copy(pids_hbm.at[pl.ds(start, CHUNK)], idx_p)
        cp_p = pltpu.async_copy(ptab_hbm.at[idx_p], buf_p, sem_p)
        cp_w.wait()
        cp_p.wait()

        def row_body(i, _):
            for k in range(HVECS):
                x = buf_p[i, pl.ds(k * LANES, LANES)]
                plsc.addupdate(buf_w.at[i, pl.ds(k * LANES, LANES)], x)
            return 0

        lax.fori_loop(0, CHUNK, row_body, 0)
        pltpu.sync_copy(buf_w, out_hbm.at[pl.ds(start, CHUNK)])
        return 0

    lax.fori_loop(0, NCHUNK, chunk_body, 0)


def _sc_gather_sum(input_ids_flat, pos_ids_flat, word_emb, pos_emb):
    mesh = plsc.VectorSubcoreMesh(core_axis_name="c", subcore_axis_name="s")
    f = pl.kernel(
        _sc_body,
        out_type=jax.ShapeDtypeStruct((TOK, HIDDEN), jnp.float32),
        mesh=mesh,
        scratch_types=[
            pltpu.VMEM((CHUNK,), jnp.int32),
            pltpu.VMEM((CHUNK,), jnp.int32),
            pltpu.VMEM((CHUNK, HIDDEN), jnp.float32),
            pltpu.VMEM((CHUNK, HIDDEN), jnp.float32),
            pltpu.SemaphoreType.DMA,
            pltpu.SemaphoreType.DMA,
        ],
    )
    return f(input_ids_flat, pos_ids_flat, word_emb, pos_emb)


LN_BLK = 512
LN_GRID = TOK // LN_BLK


def _ln_body(sum_ref, tt_ref, ip_ref, type_ref, item_ref, g_ref, b_ref,
             out_ref):
    x = sum_ref[...]
    tt = tt_ref[0, 0, :]
    ip = ip_ref[0, 0, :]
    oh_t = (tt[:, None] == lax.broadcasted_iota(jnp.int32, (LN_BLK, 4), 1)
            ).astype(jnp.float32)
    oh_i = (ip[:, None] == lax.broadcasted_iota(jnp.int32, (LN_BLK, 32), 1)
            ).astype(jnp.float32)
    x = x + jnp.dot(oh_t, type_ref[...], preferred_element_type=jnp.float32)
    x = x + jnp.dot(oh_i, item_ref[...], preferred_element_type=jnp.float32)
    mean = jnp.mean(x, axis=1, keepdims=True)
    d = x - mean
    var = jnp.mean(d * d, axis=1, keepdims=True)
    y = d * lax.rsqrt(var + EPS)
    out_ref[...] = y * g_ref[...] + b_ref[...]


def _ln(sum_wp, tt3, ip3, type_emb, item_emb, gamma2, beta2):
    return pl.pallas_call(
        _ln_body,
        grid=(LN_GRID,),
        in_specs=[
            pl.BlockSpec((LN_BLK, HIDDEN), lambda i: (i, 0)),
            pl.BlockSpec((1, 1, LN_BLK), lambda i: (i, 0, 0)),
            pl.BlockSpec((1, 1, LN_BLK), lambda i: (i, 0, 0)),
            pl.BlockSpec((4, HIDDEN), lambda i: (0, 0)),
            pl.BlockSpec((32, HIDDEN), lambda i: (0, 0)),
            pl.BlockSpec((1, HIDDEN), lambda i: (0, 0)),
            pl.BlockSpec((1, HIDDEN), lambda i: (0, 0)),
        ],
        out_specs=pl.BlockSpec((LN_BLK, HIDDEN), lambda i: (i, 0)),
        out_shape=jax.ShapeDtypeStruct((TOK, HIDDEN), jnp.float32),
    )(sum_wp, tt3, ip3, type_emb, item_emb, gamma2, beta2)


def kernel(input_ids, token_type_ids, item_position_ids, word_emb, pos_emb,
           type_emb, item_emb, ln_gamma, ln_beta):
    pos_ids = _position_ids(input_ids)
    sum_wp = _sc_gather_sum(
        input_ids.reshape(TOK), pos_ids.reshape(TOK), word_emb, pos_emb
    )
    tt3 = token_type_ids.reshape(LN_GRID, 1, LN_BLK)
    ip3 = item_position_ids.reshape(LN_GRID, 1, LN_BLK)
    out = _ln(sum_wp, tt3, ip3, type_emb, item_emb,
              ln_gamma.reshape(1, HIDDEN), ln_beta.reshape(1, HIDDEN))
    return out.reshape(B, S, HIDDEN)


# speedup vs baseline: 1.9656x; 1.0727x over previous
import functools

import jax
import jax.numpy as jnp
from jax import lax
from jax.experimental import pallas as pl
from jax.experimental.pallas import tpu as pltpu
from jax.experimental.pallas import tpu_sc as plsc

VOCAB = 50265
HIDDEN = 768
PAD = 1
EPS = 1e-12
B, S = 4, 2048
TOK = B * S

NUM_WORKERS = 32
PER_W = TOK // NUM_WORKERS
CHUNK = 32
NCHUNK = PER_W // CHUNK
LANES = 16
HVECS = HIDDEN // LANES


def _pos_ids_body(ids_ref, out_ref):
    mask = (ids_ref[...] != PAD).astype(jnp.int32)
    c = mask
    k = 1
    while k < S:
        shifted = jnp.concatenate(
            [jnp.zeros((B, k), jnp.int32), c[:, : S - k]], axis=1
        )
        c = c + shifted
        k *= 2
    out_ref[...] = c * mask + PAD


def _position_ids(input_ids):
    return pl.pallas_call(
        _pos_ids_body,
        out_shape=jax.ShapeDtypeStruct((B, S), jnp.int32),
    )(input_ids)


def _sc_body(wids_hbm, pids_hbm, wtab_hbm, ptab_hbm, out_hbm,
             idxw0, idxw1, idxp0, idxp1, bw0, bw1, bp0, bp1,
             sw0, sw1, sp0, sp1, so0, so1):
    cid = lax.axis_index("c")
    sid = lax.axis_index("s")
    wid = cid * 16 + sid
    base = wid * PER_W

    idxw = [idxw0, idxw1]
    idxp = [idxp0, idxp1]
    bw = [bw0, bw1]
    bp = [bp0, bp1]
    sw = [sw0, sw1]
    sp = [sp0, sp1]
    so = [so0, so1]

    pend_g = [None, None]
    pend_s = [None, None]
    for j in range(NCHUNK + 1):
        s = j & 1
        if j < NCHUNK:
            start = base + j * CHUNK
            if pend_s[s] is not None:
                pend_s[s].wait()
                pend_s[s] = None
            pltpu.sync_copy(wids_hbm.at[pl.ds(start, CHUNK)], idxw[s])
            pltpu.sync_copy(pids_hbm.at[pl.ds(start, CHUNK)], idxp[s])
            cw = pltpu.async_copy(wtab_hbm.at[idxw[s]], bw[s], sw[s])
            cp = pltpu.async_copy(ptab_hbm.at[idxp[s]], bp[s], sp[s])
            pend_g[s] = (cw, cp)
        if j >= 1:
            t = (j - 1) & 1
            cw, cp = pend_g[t]
            cw.wait()
            cp.wait()
            bwt, bpt = bw[t], bp[t]

            def row_body(i, _, bwt=bwt, bpt=bpt):
                for k in range(HVECS):
                    x = bpt[i, pl.ds(k * LANES, LANES)]
                    plsc.addupdate(bwt.at[i, pl.ds(k * LANES, LANES)], x)
                return 0

            lax.fori_loop(0, CHUNK, row_body, 0)
            pend_s[t] = pltpu.async_copy(
                bwt, out_hbm.at[pl.ds(base + (j - 1) * CHUNK, CHUNK)], so[t]
            )
    pend_s[0].wait()
    pend_s[1].wait()


def _sc_gather_sum(input_ids_flat, pos_ids_flat, word_emb, pos_emb):
    mesh = plsc.VectorSubcoreMesh(core_axis_name="c", subcore_axis_name="s")
    f = pl.kernel(
        _sc_body,
        out_type=jax.ShapeDtypeStruct((TOK, HIDDEN), jnp.float32),
        mesh=mesh,
        scratch_types=[
            pltpu.VMEM((CHUNK,), jnp.int32),
            pltpu.VMEM((CHUNK,), jnp.int32),
            pltpu.VMEM((CHUNK,), jnp.int32),
            pltpu.VMEM((CHUNK,), jnp.int32),
            pltpu.VMEM((CHUNK, HIDDEN), jnp.float32),
            pltpu.VMEM((CHUNK, HIDDEN), jnp.float32),
            pltpu.VMEM((CHUNK, HIDDEN), jnp.float32),
            pltpu.VMEM((CHUNK, HIDDEN), jnp.float32),
            pltpu.SemaphoreType.DMA,
            pltpu.SemaphoreType.DMA,
            pltpu.SemaphoreType.DMA,
            pltpu.SemaphoreType.DMA,
            pltpu.SemaphoreType.DMA,
            pltpu.SemaphoreType.DMA,
        ],
    )
    return f(input_ids_flat, pos_ids_flat, word_emb, pos_emb)


LN_BLK = 512
LN_GRID = TOK // LN_BLK


def _ln_body(sum_ref, tt_ref, ip_ref, type_ref, item_ref, g_ref, b_ref,
             out_ref):
    x = sum_ref[...]
    tt = tt_ref[0, 0, :]
    ip = ip_ref[0, 0, :]
    oh_t = (tt[:, None] == lax.broadcasted_iota(jnp.int32, (LN_BLK, 4), 1)
            ).astype(jnp.float32)
    oh_i = (ip[:, None] == lax.broadcasted_iota(jnp.int32, (LN_BLK, 32), 1)
            ).astype(jnp.float32)
    x = x + jnp.dot(oh_t, type_ref[...], preferred_element_type=jnp.float32)
    x = x + jnp.dot(oh_i, item_ref[...], preferred_element_type=jnp.float32)
    mean = jnp.mean(x, axis=1, keepdims=True)
    d = x - mean
    var = jnp.mean(d * d, axis=1, keepdims=True)
    y = d * lax.rsqrt(var + EPS)
    out_ref[...] = y * g_ref[...] + b_ref[...]


def _ln(sum_wp, tt3, ip3, type_emb, item_emb, gamma2, beta2):
    return pl.pallas_call(
        _ln_body,
        grid=(LN_GRID,),
        in_specs=[
            pl.BlockSpec((LN_BLK, HIDDEN), lambda i: (i, 0)),
            pl.BlockSpec((1, 1, LN_BLK), lambda i: (i, 0, 0)),
            pl.BlockSpec((1, 1, LN_BLK), lambda i: (i, 0, 0)),
            pl.BlockSpec((4, HIDDEN), lambda i: (0, 0)),
            pl.BlockSpec((32, HIDDEN), lambda i: (0, 0)),
            pl.BlockSpec((1, HIDDEN), lambda i: (0, 0)),
            pl.BlockSpec((1, HIDDEN), lambda i: (0, 0)),
        ],
        out_specs=pl.BlockSpec((LN_BLK, HIDDEN), lambda i: (i, 0)),
        out_shape=jax.ShapeDtypeStruct((TOK, HIDDEN), jnp.float32),
    )(sum_wp, tt3, ip3, type_emb, item_emb, gamma2, beta2)


def kernel(input_ids, token_type_ids, item_position_ids, word_emb, pos_emb,
           type_emb, item_emb, ln_gamma, ln_beta):
    pos_ids = _position_ids(input_ids)
    sum_wp = _sc_gather_sum(
        input_ids.reshape(TOK), pos_ids.reshape(TOK), word_emb, pos_emb
    )
    tt3 = token_type_ids.reshape(LN_GRID, 1, LN_BLK)
    ip3 = item_position_ids.reshape(LN_GRID, 1, LN_BLK)
    out = _ln(sum_wp, tt3, ip3, type_emb, item_emb,
              ln_gamma.reshape(1, HIDDEN), ln_beta.reshape(1, HIDDEN))
    return out.reshape(B, S, HIDDEN)


# speedup vs baseline: 2.1357x; 1.0865x over previous
import jax
import jax.numpy as jnp
from jax import lax
from jax.experimental import pallas as pl
from jax.experimental.pallas import tpu as pltpu
from jax.experimental.pallas import tpu_sc as plsc

VOCAB = 50265
HIDDEN = 768
PAD = 1
EPS = 1e-12
B, S = 4, 2048
TOK = B * S

NUM_WORKERS = 32
PER_W = TOK // NUM_WORKERS
CHUNK = 32
NCHUNK = PER_W // CHUNK
LANES = 16
HVECS = HIDDEN // LANES


def _aux_body(ids_ref, pos_ref, okf_ref):
    mask = (ids_ref[...] != PAD).astype(jnp.int32)
    c = mask
    k = 1
    while k < S:
        shifted = jnp.concatenate(
            [jnp.zeros((B, k), jnp.int32), c[:, : S - k]], axis=1
        )
        c = c + shifted
        k *= 2
    pos = c * mask + PAD
    exp = lax.broadcasted_iota(jnp.int32, (B, S), 1) + 2
    pos_ref[...] = pos
    okf_ref[...] = (pos == exp).astype(jnp.float32)


def _pos_aux(input_ids):
    return pl.pallas_call(
        _aux_body,
        out_shape=(
            jax.ShapeDtypeStruct((B, S), jnp.int32),
            jax.ShapeDtypeStruct((B, S), jnp.float32),
        ),
    )(input_ids)


def _sc_fast_body(wids_hbm, wtab_hbm, out_hbm,
                  idxw0, idxw1, bw0, bw1, sw0, sw1, so0, so1):
    cid = lax.axis_index("c")
    sid = lax.axis_index("s")
    wid = cid * 16 + sid
    base = wid * PER_W

    idxw = [idxw0, idxw1]
    bw = [bw0, bw1]
    sw = [sw0, sw1]
    so = [so0, so1]

    pend_g = [None, None]
    pend_s = [None, None]
    for j in range(NCHUNK + 1):
        s = j & 1
        if j < NCHUNK:
            start = base + j * CHUNK
            if pend_s[s] is not None:
                pend_s[s].wait()
                pend_s[s] = None
            pltpu.sync_copy(wids_hbm.at[pl.ds(start, CHUNK)], idxw[s])
            pend_g[s] = pltpu.async_copy(wtab_hbm.at[idxw[s]], bw[s], sw[s])
        if j >= 1:
            t = (j - 1) & 1
            pend_g[t].wait()
            pend_s[t] = pltpu.async_copy(
                bw[t], out_hbm.at[pl.ds(base + (j - 1) * CHUNK, CHUNK)], so[t]
            )
    pend_s[0].wait()
    pend_s[1].wait()


def _sc_fast(input_ids_flat, word_emb):
    mesh = plsc.VectorSubcoreMesh(core_axis_name="c", subcore_axis_name="s")
    f = pl.kernel(
        _sc_fast_body,
        out_type=jax.ShapeDtypeStruct((TOK, HIDDEN), jnp.float32),
        mesh=mesh,
        scratch_types=[
            pltpu.VMEM((CHUNK,), jnp.int32),
            pltpu.VMEM((CHUNK,), jnp.int32),
            pltpu.VMEM((CHUNK, HIDDEN), jnp.float32),
            pltpu.VMEM((CHUNK, HIDDEN), jnp.float32),
            pltpu.SemaphoreType.DMA,
            pltpu.SemaphoreType.DMA,
            pltpu.SemaphoreType.DMA,
            pltpu.SemaphoreType.DMA,
        ],
    )
    return f(input_ids_flat, word_emb)


def _sc_full_body(wids_hbm, pids_hbm, wtab_hbm, ptab_hbm, out_hbm,
                  idxw0, idxw1, idxp0, idxp1, bw0, bw1, bp0, bp1,
                  sw0, sw1, sp0, sp1, so0, so1):
    cid = lax.axis_index("c")
    sid = lax.axis_index("s")
    wid = cid * 16 + sid
    base = wid * PER_W

    idxw = [idxw0, idxw1]
    idxp = [idxp0, idxp1]
    bw = [bw0, bw1]
    bp = [bp0, bp1]
    sw = [sw0, sw1]
    sp = [sp0, sp1]
    so = [so0, so1]

    pend_g = [None, None]
    pend_s = [None, None]
    for j in range(NCHUNK + 1):
        s = j & 1
        if j < NCHUNK:
            start = base + j * CHUNK
            if pend_s[s] is not None:
                pend_s[s].wait()
                pend_s[s] = None
            pltpu.sync_copy(wids_hbm.at[pl.ds(start, CHUNK)], idxw[s])
            pltpu.sync_copy(pids_hbm.at[pl.ds(start, CHUNK)], idxp[s])
            cw = pltpu.async_copy(wtab_hbm.at[idxw[s]], bw[s], sw[s])
            cp = pltpu.async_copy(ptab_hbm.at[idxp[s]], bp[s], sp[s])
            pend_g[s] = (cw, cp)
        if j >= 1:
            t = (j - 1) & 1
            cw, cp = pend_g[t]
            cw.wait()
            cp.wait()
            bwt, bpt = bw[t], bp[t]

            def row_body(i, _, bwt=bwt, bpt=bpt):
                for k in range(HVECS):
                    x = bpt[i, pl.ds(k * LANES, LANES)]
                    plsc.addupdate(bwt.at[i, pl.ds(k * LANES, LANES)], x)
                return 0

            lax.fori_loop(0, CHUNK, row_body, 0)
            pend_s[t] = pltpu.async_copy(
                bwt, out_hbm.at[pl.ds(base + (j - 1) * CHUNK, CHUNK)], so[t]
            )
    pend_s[0].wait()
    pend_s[1].wait()


def _sc_full(input_ids_flat, pos_ids_flat, word_emb, pos_emb):
    mesh = plsc.VectorSubcoreMesh(core_axis_name="c", subcore_axis_name="s")
    f = pl.kernel(
        _sc_full_body,
        out_type=jax.ShapeDtypeStruct((TOK, HIDDEN), jnp.float32),
        mesh=mesh,
        scratch_types=[
            pltpu.VMEM((CHUNK,), jnp.int32),
            pltpu.VMEM((CHUNK,), jnp.int32),
            pltpu.VMEM((CHUNK,), jnp.int32),
            pltpu.VMEM((CHUNK,), jnp.int32),
            pltpu.VMEM((CHUNK, HIDDEN), jnp.float32),
            pltpu.VMEM((CHUNK, HIDDEN), jnp.float32),
            pltpu.VMEM((CHUNK, HIDDEN), jnp.float32),
            pltpu.VMEM((CHUNK, HIDDEN), jnp.float32),
            pltpu.SemaphoreType.DMA,
            pltpu.SemaphoreType.DMA,
            pltpu.SemaphoreType.DMA,
            pltpu.SemaphoreType.DMA,
            pltpu.SemaphoreType.DMA,
            pltpu.SemaphoreType.DMA,
        ],
    )
    return f(input_ids_flat, pos_ids_flat, word_emb, pos_emb)


SB = 512
NSB = S // SB
ROWS = B * SB


def _ln_body(sum_ref, tt_ref, ip_ref, ok_ref, type_ref, item_ref,
             g_ref, b_ref, pos_hbm, out_ref, pos_v, sem):
    i = pl.program_id(0)
    cp = pltpu.make_async_copy(pos_hbm.at[pl.ds(i * SB, SB + 8)], pos_v, sem)
    cp.start()
    tt = tt_ref[0, 0]
    ip = ip_ref[0, 0]
    oh_t = (tt[:, None] == lax.broadcasted_iota(jnp.int32, (ROWS, 4), 1)
            ).astype(jnp.float32)
    oh_i = (ip[:, None] == lax.broadcasted_iota(jnp.int32, (ROWS, 32), 1)
            ).astype(jnp.float32)
    small = jnp.dot(oh_t, type_ref[...], preferred_element_type=jnp.float32)
    small = small + jnp.dot(oh_i, item_ref[...],
                            preferred_element_type=jnp.float32)
    cp.wait()
    posd = pos_v[pl.ds(2, SB), :]
    post = jnp.concatenate([posd, posd, posd, posd], axis=0)
    x = sum_ref[...].reshape(ROWS, HIDDEN)
    x = x + ok_ref[0, 0][:, None] * post + small
    mean = jnp.mean(x, axis=1, keepdims=True)
    d = x - mean
    var = jnp.mean(d * d, axis=1, keepdims=True)
    y = d * lax.rsqrt(var + EPS)
    y = y * g_ref[...] + b_ref[...]
    out_ref[...] = y.reshape(B, SB, HIDDEN)


def _ln(sum3, tt_r, ip_r, ok_r, type_emb, item_emb, gamma2, beta2, pos_emb):
    return pl.pallas_call(
        _ln_body,
        grid=(NSB,),
        in_specs=[
            pl.BlockSpec((B, SB, HIDDEN), lambda i: (0, i, 0)),
            pl.BlockSpec((1, 1, ROWS), lambda i: (i, 0, 0)),
            pl.BlockSpec((1, 1, ROWS), lambda i: (i, 0, 0)),
            pl.BlockSpec((1, 1, ROWS), lambda i: (i, 0, 0)),
            pl.BlockSpec((4, HIDDEN), lambda i: (0, 0)),
            pl.BlockSpec((32, HIDDEN), lambda i: (0, 0)),
            pl.BlockSpec((1, HIDDEN), lambda i: (0, 0)),
            pl.BlockSpec((1, HIDDEN), lambda i: (0, 0)),
            pl.BlockSpec(memory_space=pltpu.HBM),
        ],
        out_specs=pl.BlockSpec((B, SB, HIDDEN), lambda i: (0, i, 0)),
        out_shape=jax.ShapeDtypeStruct((B, S, HIDDEN), jnp.float32),
        scratch_shapes=[
            pltpu.VMEM((SB + 8, HIDDEN), jnp.float32),
            pltpu.SemaphoreType.DMA,
        ],
    )(sum3, tt_r, ip_r, ok_r, type_emb, item_emb, gamma2, beta2, pos_emb)


def kernel(input_ids, token_type_ids, item_position_ids, word_emb, pos_emb,
           type_emb, item_emb, ln_gamma, ln_beta):
    pos_ids, okf = _pos_aux(input_ids)
    wids = input_ids.reshape(TOK)
    shifted = jnp.min(okf) < 0.5

    sum_w = lax.cond(
        shifted,
        lambda: _sc_full(wids, pos_ids.reshape(TOK), word_emb, pos_emb),
        lambda: _sc_fast(wids, word_emb),
    )
    okf_ln = okf * jnp.where(shifted, 0.0, 1.0)

    tt_r = (token_type_ids.reshape(B, NSB, SB).transpose(1, 0, 2)
            .reshape(NSB, 1, ROWS))
    ip_r = (item_position_ids.reshape(B, NSB, SB).transpose(1, 0, 2)
            .reshape(NSB, 1, ROWS))
    ok_r = okf_ln.reshape(B, NSB, SB).transpose(1, 0, 2).reshape(NSB, 1, ROWS)
    return _ln(
        sum_w.reshape(B, S, HIDDEN), tt_r, ip_r, ok_r, type_emb, item_emb,
        ln_gamma.reshape(1, HIDDEN), ln_beta.reshape(1, HIDDEN), pos_emb,
    )
